# Initial kernel scaffold; baseline (speedup 1.0000x reference)
import jax
import jax.numpy as jnp
from jax.experimental import pallas as pl


def kernel(x, mask, v):
    raise NotImplementedError("write your pallas kernel here")


# speedup vs baseline: 3.2333x; 3.2333x over previous
import functools

import jax
import jax.numpy as jnp
from jax import lax
from jax.experimental import pallas as pl
from jax.experimental.pallas import tpu as pltpu
from jax.experimental.pallas import tpu_sc as plsc

N_ITERS = 5
T = 1.0
K = 4
NW = 32
CHUNK = 128


@functools.lru_cache(maxsize=None)
def _sc_gather_fn(V, D, N):
    per_w = N // NW
    n_ch = per_w // CHUNK
    mesh = plsc.VectorSubcoreMesh(core_axis_name="c", subcore_axis_name="s")

    @functools.partial(
        pl.kernel, mesh=mesh,
        out_type=jax.ShapeDtypeStruct((N, D), jnp.float32),
        scratch_types=[
            pltpu.VMEM((CHUNK,), jnp.int32),
            pltpu.VMEM((CHUNK, D), jnp.float32),
            pltpu.SemaphoreType.DMA,
        ],
        compiler_params=pltpu.CompilerParams(use_tc_tiling_on_sc=False),
    )
    def k(table_hbm, idx_hbm, out_hbm, idx_v, rows_v, sem):
        wid = lax.axis_index("s") * 2 + lax.axis_index("c")
        base = wid * per_w

        def body(i, carry):
            off = base + i * CHUNK
            pltpu.sync_copy(idx_hbm.at[pl.ds(off, CHUNK)], idx_v)
            pltpu.async_copy(table_hbm.at[idx_v], rows_v, sem).wait()
            pltpu.sync_copy(rows_v, out_hbm.at[pl.ds(off, CHUNK)])
            return carry

        lax.fori_loop(0, n_ch, body, 0)

    return k


def _sc_gather(table, idx):
    V, D = table.shape
    (N,) = idx.shape
    return _sc_gather_fn(V, D, N)(table, idx)


def _box3(x):
    _, H, W = x.shape
    xp = jnp.pad(x, ((0, 0), (1, 1), (1, 1)))
    out = jnp.zeros_like(x)
    for di in range(3):
        for dj in range(3):
            out = out + xp[:, di:di + H, dj:dj + W]
    return out


def _patch_cost(a_rows, a_img, pen_flat, si, sj, H, W):
    Kp = si.shape[0]
    lin = (si * W + sj).reshape(Kp * H * W)
    g = _sc_gather(a_rows, lin)
    g = g.reshape(Kp, H * W, -1)
    d = jnp.sum((g - a_img[None]) ** 2, axis=-1)
    d = _box3(d.reshape(Kp, H, W))
    pen = pen_flat[lin].reshape(Kp, H, W)
    return d + pen


def _one_image(a, m, v, key):
    C, H, W = a.shape
    a_rows = a.reshape(C, H * W).T
    v_rows = v.reshape(C, H * W).T
    pen_flat = (1.0 - m.reshape(H * W)) * 1e6

    k1, k2 = jax.random.split(key)
    si = jax.random.randint(k1, (K, H, W), 0, H)
    sj = jax.random.randint(k2, (K, H, W), 0, W)
    cost = _patch_cost(a_rows, a_rows, pen_flat, si, sj, H, W)

    for t in range(N_ITERS):
        kt = jax.random.fold_in(key, t + 7)
        ka, kb = jax.random.split(kt)
        si_d = jnp.clip(jnp.roll(si, 1, axis=1) + 1, 0, H - 1)
        sj_d = jnp.roll(sj, 1, axis=1)
        si_r = jnp.roll(si, 1, axis=2)
        sj_r = jnp.clip(jnp.roll(sj, 1, axis=2) + 1, 0, W - 1)
        si_n = jax.random.randint(ka, (K, H, W), 0, H)
        sj_n = jax.random.randint(kb, (K, H, W), 0, W)
        cand_si = jnp.concatenate([si, si_d, si_r, si_n], axis=0)
        cand_sj = jnp.concatenate([sj, sj_d, sj_r, sj_n], axis=0)
        cand_cost = jnp.concatenate([
            cost,
            _patch_cost(a_rows, a_rows, pen_flat, si_d, sj_d, H, W),
            _patch_cost(a_rows, a_rows, pen_flat, si_r, sj_r, H, W),
            _patch_cost(a_rows, a_rows, pen_flat, si_n, sj_n, H, W),
        ], axis=0)
        neg = -jnp.moveaxis(cand_cost, 0, -1)
        _, idx = jax.lax.top_k(neg, K)
        idx = jnp.moveaxis(idx, -1, 0)
        si = jnp.take_along_axis(cand_si, idx, axis=0)
        sj = jnp.take_along_axis(cand_sj, idx, axis=0)
        cost = jnp.take_along_axis(cand_cost, idx, axis=0)

    cost_map = _patch_cost(a_rows, a_rows, pen_flat, si, sj, H, W)
    w = jax.nn.softmax(-cost_map / T, axis=0)

    lin = (si * W + sj).reshape(K * H * W)
    gv = _sc_gather(v_rows, lin).reshape(K, H * W, C)
    out = jnp.sum(w.reshape(K, H * W, 1) * gv, axis=0)
    return out.T.reshape(C, H, W)


def kernel(x, mask, v):
    mb = (mask > 0.5).astype(x.dtype)
    outs = []
    for i in range(x.shape[0]):
        key = jax.random.fold_in(jax.random.key(42), i)
        outs.append(_one_image(x[i], mb[i], v[i], key))
    return jnp.stack(outs, axis=0)


# speedup vs baseline: 15.4068x; 4.7651x over previous
import functools

import jax
import jax.numpy as jnp
from jax import lax
from jax.experimental import pallas as pl
from jax.experimental.pallas import tpu as pltpu
from jax.experimental.pallas import tpu_sc as plsc

N_ITERS = 5
T = 1.0
K = 4
NW = 32
CHUNK = 128


@functools.lru_cache(maxsize=None)
def _sc_gather_fn(V, D, N):
    per_w = N // NW
    n_ch = per_w // CHUNK
    mesh = plsc.VectorSubcoreMesh(core_axis_name="c", subcore_axis_name="s")

    @functools.partial(
        pl.kernel, mesh=mesh,
        out_type=jax.ShapeDtypeStruct((N, D), jnp.float32),
        scratch_types=[
            pltpu.VMEM((CHUNK,), jnp.int32),
            pltpu.VMEM((CHUNK, D), jnp.float32),
            pltpu.SemaphoreType.DMA,
        ],
        compiler_params=pltpu.CompilerParams(use_tc_tiling_on_sc=False,
                                             needs_layout_passes=False),
    )
    def k(table_hbm, idx_hbm, out_hbm, idx_v, rows_v, sem):
        wid = lax.axis_index("s") * 2 + lax.axis_index("c")
        base = wid * per_w

        def body(i, carry):
            off = base + i * CHUNK
            pltpu.sync_copy(idx_hbm.at[pl.ds(off, CHUNK)], idx_v)
            pltpu.async_copy(table_hbm.at[idx_v], rows_v, sem).wait()
            pltpu.sync_copy(rows_v, out_hbm.at[pl.ds(off, CHUNK)])
            return carry

        lax.fori_loop(0, n_ch, body, 0)

    return k


def _sc_gather(table, idx):
    V, D = table.shape
    (N,) = idx.shape
    return _sc_gather_fn(V, D, N)(table, idx)


PIX_CH = 112


@functools.lru_cache(maxsize=None)
def _sc_dist_fn(HW, C, Kp):
    per_w = HW // NW
    n_ch = per_w // PIX_CH
    n_g16 = PIX_CH // 16
    mesh = plsc.VectorSubcoreMesh(core_axis_name="c", subcore_axis_name="s")

    @functools.partial(
        pl.kernel, mesh=mesh,
        out_type=(jax.ShapeDtypeStruct((Kp, HW), jnp.float32),
                  jax.ShapeDtypeStruct((Kp, HW), jnp.float32)),
        scratch_types=[
            pltpu.VMEM((Kp, PIX_CH, C), jnp.float32),
            pltpu.VMEM((C, PIX_CH), jnp.float32),
            pltpu.VMEM((HW,), jnp.float32),
            pltpu.VMEM((Kp, PIX_CH), jnp.int32),
            pltpu.VMEM((Kp, PIX_CH), jnp.float32),
            pltpu.VMEM((Kp, PIX_CH), jnp.float32),
            pltpu.SemaphoreType.DMA,
        ],
        compiler_params=pltpu.CompilerParams(use_tc_tiling_on_sc=False,
                                             needs_layout_passes=False),
    )
    def k(tbl_hbm, tT_hbm, pen_hbm, idx_hbm, d_out, pen_out,
          g_v, aq_v, penall_v, idx_v, d_v, pend_v, sem):
        wid = lax.axis_index("s") * 2 + lax.axis_index("c")
        base = wid * per_w
        pltpu.sync_copy(pen_hbm, penall_v)

        def chunk(i, carry):
            off = base + i * PIX_CH
            pltpu.sync_copy(idx_hbm.at[:, pl.ds(off, PIX_CH)], idx_v)
            copies = [
                pltpu.async_copy(tbl_hbm.at[idx_v.at[kk]], g_v.at[kk], sem)
                for kk in range(Kp)
            ]
            pltpu.sync_copy(tT_hbm.at[:, pl.ds(off, PIX_CH)], aq_v)
            for cp in copies:
                cp.wait()
            lanes = lax.iota(jnp.int32, 16)
            for g16 in range(n_g16):
                p0 = g16 * 16
                prow = p0 + lanes
                for kk in range(Kp):
                    ksplat = jnp.full((16,), kk, jnp.int32)

                    def grp8(r, accs, ksplat=ksplat):
                        new = []
                        for s in range(8):
                            c = r * 8 + s
                            aq = aq_v[c, pl.ds(p0, 16)]
                            gk = plsc.load_gather(
                                g_v, [ksplat, prow,
                                      jnp.full((16,), s, jnp.int32) + r * 8])
                            t = gk - aq
                            new.append(accs[s] + t * t)
                        return tuple(new)

                    accs = list(lax.fori_loop(
                        1, C // 8, grp8,
                        grp8(0, tuple(jnp.zeros((16,), jnp.float32)
                                      for _ in range(8)))))
                    while len(accs) > 1:
                        h = len(accs) // 2
                        accs = [accs[s] + accs[s + h] for s in range(h)]
                    d_v[kk, pl.ds(p0, 16)] = accs[0]
                    iv = idx_v[kk, pl.ds(p0, 16)]
                    pend_v[kk, pl.ds(p0, 16)] = plsc.load_gather(penall_v, [iv])
            pltpu.sync_copy(d_v, d_out.at[:, pl.ds(off, PIX_CH)])
            pltpu.sync_copy(pend_v, pen_out.at[:, pl.ds(off, PIX_CH)])
            return carry

        lax.fori_loop(0, n_ch, chunk, 0)

    return k


def _box3(x):
    _, H, W = x.shape
    xp = jnp.pad(x, ((0, 0), (1, 1), (1, 1)))
    out = jnp.zeros_like(x)
    for di in range(3):
        for dj in range(3):
            out = out + xp[:, di:di + H, dj:dj + W]
    return out


def _patch_cost(a_rows, a_T, pen_flat, si, sj, H, W):
    Kp, C = si.shape[0], a_rows.shape[1]
    lin = (si * W + sj).reshape(Kp, H * W)
    d, pen = _sc_dist_fn(H * W, C, Kp)(a_rows, a_T, pen_flat, lin)
    d = _box3(d.reshape(Kp, H, W))
    return d + pen.reshape(Kp, H, W)


def _one_image(a, m, v, key):
    C, H, W = a.shape
    a_T = a.reshape(C, H * W)
    a_rows = a_T.T
    v_rows = v.reshape(C, H * W).T
    pen_flat = (1.0 - m.reshape(H * W)) * 1e6

    k1, k2 = jax.random.split(key)
    si = jax.random.randint(k1, (K, H, W), 0, H)
    sj = jax.random.randint(k2, (K, H, W), 0, W)
    cost = _patch_cost(a_rows, a_T, pen_flat, si, sj, H, W)

    for t in range(N_ITERS):
        kt = jax.random.fold_in(key, t + 7)
        ka, kb = jax.random.split(kt)
        si_d = jnp.clip(jnp.roll(si, 1, axis=1) + 1, 0, H - 1)
        sj_d = jnp.roll(sj, 1, axis=1)
        si_r = jnp.roll(si, 1, axis=2)
        sj_r = jnp.clip(jnp.roll(sj, 1, axis=2) + 1, 0, W - 1)
        si_n = jax.random.randint(ka, (K, H, W), 0, H)
        sj_n = jax.random.randint(kb, (K, H, W), 0, W)
        cand_si = jnp.concatenate([si, si_d, si_r, si_n], axis=0)
        cand_sj = jnp.concatenate([sj, sj_d, sj_r, sj_n], axis=0)
        cand_cost = jnp.concatenate([
            cost,
            _patch_cost(a_rows, a_T, pen_flat, si_d, sj_d, H, W),
            _patch_cost(a_rows, a_T, pen_flat, si_r, sj_r, H, W),
            _patch_cost(a_rows, a_T, pen_flat, si_n, sj_n, H, W),
        ], axis=0)
        neg = -jnp.moveaxis(cand_cost, 0, -1)
        _, idx = jax.lax.top_k(neg, K)
        idx = jnp.moveaxis(idx, -1, 0)
        si = jnp.take_along_axis(cand_si, idx, axis=0)
        sj = jnp.take_along_axis(cand_sj, idx, axis=0)
        cost = jnp.take_along_axis(cand_cost, idx, axis=0)

    cost_map = _patch_cost(a_rows, a_T, pen_flat, si, sj, H, W)
    w = jax.nn.softmax(-cost_map / T, axis=0)

    lin = (si * W + sj).reshape(K * H * W)
    gv = _sc_gather(v_rows, lin).reshape(K, H * W, C)
    out = jnp.sum(w.reshape(K, H * W, 1) * gv, axis=0)
    return out.T.reshape(C, H, W)


def kernel(x, mask, v):
    mb = (mask > 0.5).astype(x.dtype)
    outs = []
    for i in range(x.shape[0]):
        key = jax.random.fold_in(jax.random.key(42), i)
        outs.append(_one_image(x[i], mb[i], v[i], key))
    return jnp.stack(outs, axis=0)


# speedup vs baseline: 15.4511x; 1.0029x over previous
import functools

import jax
import jax.numpy as jnp
from jax import lax
from jax.experimental import pallas as pl
from jax.experimental.pallas import tpu as pltpu
from jax.experimental.pallas import tpu_sc as plsc

N_ITERS = 5
T = 1.0
K = 4
NW = 32
PIX = 32

_SC_PARAMS = pltpu.CompilerParams(use_tc_tiling_on_sc=False,
                                  needs_layout_passes=False)
_MESH = plsc.VectorSubcoreMesh(core_axis_name="c", subcore_axis_name="s")


def _ssd_16px(g_v, aq_v, kk, prow, p0, C):
    ksplat = jnp.zeros((16,), jnp.int32) + kk

    def grp8(r, accs):
        new = []
        for s in range(8):
            c = r * 8 + s
            aq = aq_v[c, pl.ds(p0, 16)]
            gk = plsc.load_gather(
                g_v, [ksplat, prow, jnp.full((16,), s, jnp.int32) + r * 8])
            t = gk - aq
            new.append(accs[s] + t * t)
        return tuple(new)

    accs = list(lax.fori_loop(
        1, C // 8, grp8,
        grp8(0, tuple(jnp.zeros((16,), jnp.float32) for _ in range(8)))))
    while len(accs) > 1:
        h = len(accs) // 2
        accs = [accs[s] + accs[s + h] for s in range(h)]
    return accs[0]


@functools.lru_cache(maxsize=None)
def _sc_dist_fn(HWT, C, Kp):
    per_w = HWT // NW
    n_ch = per_w // PIX
    assert n_ch % 2 == 0 and PIX % 16 == 0
    n_g16 = PIX // 16

    scr = []
    for _ in range(2):
        scr += [pltpu.VMEM((Kp, PIX), jnp.int32),
                pltpu.VMEM((Kp, PIX, C), jnp.float32),
                pltpu.VMEM((Kp, PIX, 16), jnp.float32),
                pltpu.VMEM((C, PIX), jnp.float32),
                pltpu.VMEM((Kp, PIX), jnp.float32),
                pltpu.VMEM((Kp, PIX), jnp.float32),
                pltpu.SemaphoreType.DMA,
                pltpu.SemaphoreType.DMA]

    @functools.partial(
        pl.kernel, mesh=_MESH,
        out_type=(jax.ShapeDtypeStruct((Kp, HWT), jnp.float32),
                  jax.ShapeDtypeStruct((Kp, HWT), jnp.float32)),
        scratch_types=scr,
        compiler_params=_SC_PARAMS,
    )
    def k(tbl_hbm, tT_hbm, pen_hbm, idx_hbm, d_out, pen_out, *banks):
        wid = lax.axis_index("s") * 2 + lax.axis_index("c")
        base = wid * per_w
        bank = [banks[0:8], banks[8:16]]

        def start(ci, b):
            idx_b, g_b, pg_b, aq_b, _, _, sem, _ = bank[b]
            off = base + ci * PIX
            pltpu.sync_copy(idx_hbm.at[:, pl.ds(off, PIX)], idx_b)
            for kk in range(Kp):
                pltpu.async_copy(tbl_hbm.at[idx_b.at[kk]], g_b.at[kk], sem)
                pltpu.async_copy(pen_hbm.at[idx_b.at[kk]], pg_b.at[kk], sem)
            pltpu.async_copy(tT_hbm.at[:, pl.ds(off, PIX)], aq_b, sem)

        def wait_in(b):
            _, g_b, pg_b, aq_b, _, _, sem, _ = bank[b]
            for kk in range(Kp):
                pltpu.make_async_copy(tbl_hbm.at[pl.ds(0, PIX)],
                                      g_b.at[kk], sem).wait()
                pltpu.make_async_copy(pen_hbm.at[pl.ds(0, PIX)],
                                      pg_b.at[kk], sem).wait()
            pltpu.make_async_copy(tT_hbm.at[:, pl.ds(0, PIX)], aq_b,
                                  sem).wait()

        def drain_out(b):
            _, _, _, _, d_b, pd_b, _, osem = bank[b]
            pltpu.make_async_copy(d_b, d_out.at[:, pl.ds(0, PIX)],
                                  osem).wait()
            pltpu.make_async_copy(pd_b, pen_out.at[:, pl.ds(0, PIX)],
                                  osem).wait()

        def compute(ci, b, drain):
            idx_b, g_b, pg_b, aq_b, d_b, pd_b, _, osem = bank[b]
            wait_in(b)
            if drain is not None:
                if drain is True:
                    drain_out(b)
                else:
                    pl.when(drain)(lambda: drain_out(b))
            off = base + ci * PIX
            lanes = lax.iota(jnp.int32, 16)
            zf = jnp.zeros((16,), jnp.int32)
            for g16 in range(n_g16):
                p0 = g16 * 16
                prow = p0 + lanes

                def permap(kk, carry2, prow=prow, p0=p0):
                    d_b[kk, pl.ds(p0, 16)] = _ssd_16px(
                        g_b, aq_b, kk, prow, p0, C)
                    pd_b[kk, pl.ds(p0, 16)] = plsc.load_gather(
                        pg_b, [jnp.zeros((16,), jnp.int32) + kk, prow, zf])
                    return carry2

                lax.fori_loop(0, Kp, permap, 0)
            pltpu.async_copy(d_b, d_out.at[:, pl.ds(off, PIX)], osem)
            pltpu.async_copy(pd_b, pen_out.at[:, pl.ds(off, PIX)], osem)

        start(0, 0)

        def body(j, carry):
            c0 = j * 2
            start(c0 + 1, 1)
            compute(c0, 0, j > 0)
            start(c0 + 2, 0)
            compute(c0 + 1, 1, j > 0)
            return carry

        lax.fori_loop(0, n_ch // 2 - 1, body, 0)
        c0 = n_ch - 2
        start(c0 + 1, 1)
        compute(c0, 0, True)
        compute(c0 + 1, 1, True)
        drain_out(0)
        drain_out(1)

    return k


@functools.lru_cache(maxsize=None)
def _sc_recon_fn(HWT, C, Kp):
    per_w = HWT // NW
    n_ch = per_w // PIX
    n_g16 = PIX // 16

    @functools.partial(
        pl.kernel, mesh=_MESH,
        out_type=jax.ShapeDtypeStruct((C, HWT), jnp.float32),
        scratch_types=[
            pltpu.VMEM((Kp, PIX), jnp.int32),
            pltpu.VMEM((Kp, PIX, C), jnp.float32),
            pltpu.VMEM((Kp, PIX), jnp.float32),
            pltpu.VMEM((C, PIX), jnp.float32),
            pltpu.SemaphoreType.DMA,
        ],
        compiler_params=_SC_PARAMS,
    )
    def k(vtbl_hbm, cost_hbm, idx_hbm, out_hbm, idx_v, g_v, cost_v, o_v, sem):
        wid = lax.axis_index("s") * 2 + lax.axis_index("c")
        base = wid * per_w

        def chunk(ci, carry):
            off = base + ci * PIX
            pltpu.sync_copy(idx_hbm.at[:, pl.ds(off, PIX)], idx_v)
            cps = [pltpu.async_copy(vtbl_hbm.at[idx_v.at[kk]], g_v.at[kk],
                                    sem) for kk in range(Kp)]
            pltpu.sync_copy(cost_hbm.at[:, pl.ds(off, PIX)], cost_v)
            for cp in cps:
                cp.wait()
            lanes = lax.iota(jnp.int32, 16)
            for g16 in range(n_g16):
                p0 = g16 * 16
                prow = p0 + lanes
                cs = [cost_v[kk, pl.ds(p0, 16)] for kk in range(Kp)]
                logits = [-c / T for c in cs]
                m = logits[0]
                for l in logits[1:]:
                    m = jnp.maximum(m, l)
                es = [jnp.exp(l - m) for l in logits]
                ssum = es[0]
                for e in es[1:]:
                    ssum = ssum + e
                ws = [e / ssum for e in es]

                def chan(c, carry2, ws=ws, prow=prow, p0=p0):
                    csplat = jnp.full((16,), 0, jnp.int32) + c
                    acc = jnp.zeros((16,), jnp.float32)
                    for kk in range(Kp):
                        gk = plsc.load_gather(
                            g_v, [jnp.full((16,), kk, jnp.int32), prow,
                                  csplat])
                        acc = acc + ws[kk] * gk
                    o_v[c, pl.ds(p0, 16)] = acc
                    return carry2

                lax.fori_loop(0, C, chan, 0)
            pltpu.sync_copy(o_v, out_hbm.at[:, pl.ds(off, PIX)])
            return carry

        lax.fori_loop(0, n_ch, chunk, 0)

    return k


def _box3(x):
    _, H, W = x.shape
    xp = jnp.pad(x, ((0, 0), (1, 1), (1, 1)))
    out = jnp.zeros_like(x)
    for di in range(3):
        for dj in range(3):
            out = out + xp[:, di:di + H, dj:dj + W]
    return out


def _patch_cost_b(tbl, tT, pen_tbl, si, sj, H, W):
    B, Kp = si.shape[0], si.shape[1]
    HW = H * W
    lin = si * W + sj + (jnp.arange(B, dtype=si.dtype) * HW)[:, None, None, None]
    lin = lin.reshape(B, Kp, HW).transpose(1, 0, 2).reshape(Kp, B * HW)
    d, pen = _sc_dist_fn(B * HW, tbl.shape[1], Kp)(tbl, tT, pen_tbl, lin)
    d = d.reshape(Kp, B, H, W).transpose(1, 0, 2, 3).reshape(B * Kp, H, W)
    d = _box3(d).reshape(B, Kp, H, W)
    pen = pen.reshape(Kp, B, H, W).transpose(1, 0, 2, 3)
    return d + pen


def kernel(x, mask, v):
    B, C, H, W = x.shape
    HW = H * W
    mb = (mask > 0.5).astype(x.dtype)

    a_T = x.transpose(1, 0, 2, 3).reshape(C, B * HW)
    tbl = a_T.T
    v_tbl = v.transpose(1, 0, 2, 3).reshape(C, B * HW).T
    pen_flat = ((1.0 - mb.reshape(B * HW)) * 1e6)
    pen_tbl = jnp.zeros((B * HW, 16), jnp.float32).at[:, 0].set(pen_flat)

    keys = [jax.random.fold_in(jax.random.key(42), i) for i in range(B)]
    si, sj = [], []
    for i in range(B):
        k1, k2 = jax.random.split(keys[i])
        si.append(jax.random.randint(k1, (K, H, W), 0, H))
        sj.append(jax.random.randint(k2, (K, H, W), 0, W))
    si = jnp.stack(si)
    sj = jnp.stack(sj)
    cost = _patch_cost_b(tbl, a_T, pen_tbl, si, sj, H, W)

    for t in range(N_ITERS):
        si_n, sj_n = [], []
        for i in range(B):
            kt = jax.random.fold_in(keys[i], t + 7)
            ka, kb = jax.random.split(kt)
            si_n.append(jax.random.randint(ka, (K, H, W), 0, H))
            sj_n.append(jax.random.randint(kb, (K, H, W), 0, W))
        si_n = jnp.stack(si_n)
        sj_n = jnp.stack(sj_n)
        si_d = jnp.clip(jnp.roll(si, 1, axis=2) + 1, 0, H - 1)
        sj_d = jnp.roll(sj, 1, axis=2)
        si_r = jnp.roll(si, 1, axis=3)
        sj_r = jnp.clip(jnp.roll(sj, 1, axis=3) + 1, 0, W - 1)
        cand_si = jnp.concatenate([si, si_d, si_r, si_n], axis=1)
        cand_sj = jnp.concatenate([sj, sj_d, sj_r, sj_n], axis=1)
        new_cost = _patch_cost_b(
            tbl, a_T, pen_tbl,
            jnp.concatenate([si_d, si_r, si_n], axis=1),
            jnp.concatenate([sj_d, sj_r, sj_n], axis=1), H, W)
        cand_cost = jnp.concatenate([cost, new_cost], axis=1)
        neg = -jnp.moveaxis(cand_cost, 1, -1)
        _, idx = jax.lax.top_k(neg, K)
        idx = jnp.moveaxis(idx, -1, 1)
        si = jnp.take_along_axis(cand_si, idx, axis=1)
        sj = jnp.take_along_axis(cand_sj, idx, axis=1)
        cost = jnp.take_along_axis(cand_cost, idx, axis=1)

    cost_map = _patch_cost_b(tbl, a_T, pen_tbl, si, sj, H, W)

    lin = si * W + sj + (jnp.arange(B, dtype=si.dtype) * HW)[:, None, None, None]
    lin = lin.reshape(B, K, HW).transpose(1, 0, 2).reshape(K, B * HW)
    cost_t = cost_map.reshape(B, K, HW).transpose(1, 0, 2).reshape(K, B * HW)
    out = _sc_recon_fn(B * HW, C, K)(v_tbl, cost_t, lin)
    return out.reshape(C, B, H, W).transpose(1, 0, 2, 3)
